# baseline (speedup 1.0000x reference)
import functools

import jax
import jax.numpy as jnp
from jax import lax
from jax.experimental import pallas as pl
from jax.experimental.pallas import tpu as pltpu
from jax.experimental.pallas import tpu_sc as plsc

N = 10000
E = 160000
G = 512
H = 512
L = 128
DEC = 512
EPS = 1e-5

NSUB = 16
ROWS_PER_SUB = 632
NP = NSUB * ROWS_PER_SUB
K = 112


def _spmm_body(n_items, vdiv, tdv, nch, wmode, dup,
               tables, packedp, wp, zeros, out,
               acc, packed_buf, w_buf, r0, r1,
               s0, s1, d0, d1, gsem0, gsem1):
    cid = lax.axis_index("c")
    sid = lax.axis_index("s")
    widx = sid if wmode == 16 else cid * 16 + sid
    rows = (r0, r1)
    sidx = (s0, s1)
    didx = (d0, d1)
    gsem = (gsem0, gsem1)

    def gather_start(b):
        pltpu.async_copy(tables.at[sidx[b]], rows[b], gsem[b])

    def gather_wait(b):
        pltpu.make_async_copy(tables.at[sidx[b]], rows[b], gsem[b]).wait()

    def item_step(t, carry):
        i = 2 * t + cid
        v = i // vdiv
        tbase = (i // tdv) * N + (cid * (3 * N) if dup else 0)
        pltpu.sync_copy(zeros, acc.at[pl.ds(sid * ROWS_PER_SUB, ROWS_PER_SUB)])
        pltpu.sync_copy(packedp.at[v, widx], packed_buf)
        pltpu.sync_copy(wp.at[v, widx], w_buf)
        plsc.subcore_barrier()

        def unpack(j, b):
            for tt in range(7):
                lane0 = K * j + 16 * tt
                q = lane0 // 128
                off = lane0 % 128
                pv = packed_buf[q, pl.ds(off, 16)]
                ol = pl.ds(tt * 16, 16)
                sidx[b][ol] = (pv & 0xFFFF) + tbase
                didx[b][ol] = pv >> 16

        def scale(j, b):
            def grp(tt, c2):
                lane0 = K * j + 16 * tt
                q = lane0 // 128
                off = lane0 % 128
                wv = w_buf[q, pl.ds(off, 16)]
                for l in range(16):
                    wk = wv[l]
                    row = 16 * tt + l
                    for cc in range(8):
                        sl = pl.ds(cc * 16, 16)
                        rows[b][row, sl] = rows[b][row, sl] * wk
                return c2
            lax.fori_loop(0, 7, grp, 0)

        unpack(0, 0)
        gather_start(0)

        def pipe(tc, c):
            for b in range(2):
                j = 2 * tc + b
                gather_wait(b)

                @pl.when(j + 1 < nch)
                def _():
                    unpack(j + 1, 1 - b)
                    gather_start(1 - b)
                scale(j, b)
                pltpu.sync_copy(rows[b], acc.at[didx[b]], add=True)
            return c
        lax.fori_loop(0, nch // 2, pipe, 0)
        plsc.subcore_barrier()
        pltpu.sync_copy(
            acc.at[pl.ds(sid * ROWS_PER_SUB, ROWS_PER_SUB)],
            out.at[pl.ds(i * NP + sid * ROWS_PER_SUB, ROWS_PER_SUB)])
        return carry

    lax.fori_loop(0, n_items // 2, item_step, 0)


def _make_spmm(n_items, vdiv, tdv, nch, nrow, wmode, dup):
    body = functools.partial(_spmm_body, n_items, vdiv, tdv, nch, wmode, dup)
    return pl.kernel(
        body,
        out_type=jax.ShapeDtypeStruct((n_items * NP, 128), jnp.float32),
        mesh=plsc.VectorSubcoreMesh(core_axis_name="c", subcore_axis_name="s"),
        scratch_types=[
            pltpu.VMEM_SHARED((NP, 128), jnp.float32),
            pltpu.VMEM((nrow, 128), jnp.int32),
            pltpu.VMEM((nrow, 128), jnp.float32),
            pltpu.VMEM((K, 128), jnp.float32),
            pltpu.VMEM((K, 128), jnp.float32),
            pltpu.VMEM((K,), jnp.int32),
            pltpu.VMEM((K,), jnp.int32),
            pltpu.VMEM((K,), jnp.int32),
            pltpu.VMEM((K,), jnp.int32),
            pltpu.SemaphoreType.DMA,
            pltpu.SemaphoreType.DMA,
        ],
    )


def _pad_edges(ei_list, w_list, nworkers, nch, nrow):
    per = E // nworkers
    pedges = nch * K
    pad = pedges - per
    words = nrow * 128
    packs, wvs = [], []
    spread = (jnp.arange(pad, dtype=jnp.int32) * 389) % N
    for ei, w in zip(ei_list, w_list):
        s = ei[0].reshape(nworkers, per).astype(jnp.int32)
        d = ei[1].reshape(nworkers, per).astype(jnp.int32)
        wv = w.reshape(nworkers, per)
        s = jnp.pad(s, ((0, 0), (0, words - per)))
        d = jnp.concatenate(
            [d, jnp.broadcast_to(spread, (nworkers, pad)),
             jnp.zeros((nworkers, words - pedges), jnp.int32)], axis=1)
        wv = jnp.pad(wv, ((0, 0), (0, words - per)))
        packs.append((s | (d << 16)).reshape(nworkers, nrow, 128))
        wvs.append(wv.reshape(nworkers, nrow, 128))
    return jnp.stack(packs), jnp.stack(wvs).astype(jnp.float32)


BN_ROWS = 1000


def _enc1_body(x_ref, w_ref, b_ref, out_ref):
    out_ref[...] = (
        jax.lax.dot_general(x_ref[...], w_ref[...], (((1,), (0,)), ((), ())),
                            preferred_element_type=jnp.float32)
        + b_ref[0])[None]


def _enc1(x, w1cat, b1cat):
    return pl.pallas_call(
        _enc1_body,
        grid=(12, N // BN_ROWS),
        in_specs=[
            pl.BlockSpec((BN_ROWS, G), lambda i, n: (n, 0)),
            pl.BlockSpec((G, 128), lambda i, n: (0, i)),
            pl.BlockSpec((1, 1, 128), lambda i, n: (i, 0, 0)),
        ],
        out_specs=pl.BlockSpec((1, BN_ROWS, 128), lambda i, n: (i, n, 0)),
        out_shape=jax.ShapeDtypeStruct((12, N, 128), jnp.float32),
    )(x, w1cat, b1cat.reshape(12, 1, 128))


def _enc2_body(s1_ref, w2_ref, b2_ref, out_ref):
    acc = b2_ref[0].astype(jnp.float32)
    for c in range(4):
        acc = acc + jax.lax.dot_general(
            jax.nn.relu(s1_ref[c]), w2_ref[0, c],
            (((1,), (0,)), ((), ())), preferred_element_type=jnp.float32)
    out_ref[...] = acc[None]


def _enc2(s1, w2s, b2s):
    return pl.pallas_call(
        _enc2_body,
        grid=(3, N // BN_ROWS),
        in_specs=[
            pl.BlockSpec((4, BN_ROWS, 128), lambda v, n: (v, n, 0)),
            pl.BlockSpec((1, 4, 128, 128), lambda v, n: (v, 0, 0, 0)),
            pl.BlockSpec((1, 1, 128), lambda v, n: (v, 0, 0)),
        ],
        out_specs=pl.BlockSpec((1, BN_ROWS, 128), lambda v, n: (v, n, 0)),
        out_shape=jax.ShapeDtypeStruct((3, N, 128), jnp.float32),
    )(s1, w2s, b2s)


def _fuse_body(s2_ref, fw_ref, fb_ref, dw_ref, db_ref,
               cw1_ref, cb1_ref, cw2_ref, cb2_ref,
               z3_ref, h1_ref, sum_ref, ssq_ref, zf_ref, zn_ref):
    zs = []
    facc = fb_ref[...].astype(jnp.float32)
    for v in range(3):
        zv = s2_ref[2 * v] + s2_ref[2 * v + 1]
        zs.append(zv)
        z3_ref[v] = zv
        facc = facc + jax.lax.dot_general(
            zv, fw_ref[v], (((1,), (0,)), ((), ())),
            preferred_element_type=jnp.float32)
    z_fused = jax.nn.relu(facc)
    h1 = jax.nn.relu(
        jax.lax.dot_general(z_fused, dw_ref[...], (((1,), (0,)), ((), ())),
                            preferred_element_type=jnp.float32) + db_ref[...])
    h1_ref[...] = h1

    @pl.when(pl.program_id(0) == 0)
    def _():
        sum_ref[...] = jnp.zeros_like(sum_ref)
        ssq_ref[...] = jnp.zeros_like(ssq_ref)
    sum_ref[...] += jnp.sum(h1, axis=0, keepdims=True)
    ssq_ref[...] += jnp.sum(h1 * h1, axis=0, keepdims=True)

    hc = jax.nn.relu(
        jax.lax.dot_general(zs[0], cw1_ref[0], (((1,), (0,)), ((), ())),
                            preferred_element_type=jnp.float32)
        + jax.lax.dot_general(zs[1], cw1_ref[1], (((1,), (0,)), ((), ())),
                              preferred_element_type=jnp.float32)
        + cb1_ref[...])
    zf = jax.lax.dot_general(hc, cw2_ref[...], (((1,), (0,)), ((), ())),
                             preferred_element_type=jnp.float32) + cb2_ref[...]
    zf_ref[...] = zf
    nrm = jnp.sqrt(jnp.sum(zf * zf, axis=1, keepdims=True))
    zn_ref[...] = zf / jnp.maximum(nrm, 1e-12)


def _fuse(s2, fw3, fb, dw1, db1, cw1, cb1, cw2, cb2):
    return pl.pallas_call(
        _fuse_body,
        grid=(N // BN_ROWS,),
        in_specs=[
            pl.BlockSpec((6, BN_ROWS, 128), lambda n: (0, n, 0)),
            pl.BlockSpec((3, 128, 128), lambda n: (0, 0, 0)),
            pl.BlockSpec((1, 128), lambda n: (0, 0)),
            pl.BlockSpec((128, DEC), lambda n: (0, 0)),
            pl.BlockSpec((1, DEC), lambda n: (0, 0)),
            pl.BlockSpec((2, 128, 128), lambda n: (0, 0, 0)),
            pl.BlockSpec((1, 128), lambda n: (0, 0)),
            pl.BlockSpec((128, 128), lambda n: (0, 0)),
            pl.BlockSpec((1, 128), lambda n: (0, 0)),
        ],
        out_specs=[
            pl.BlockSpec((3, BN_ROWS, 128), lambda n: (0, n, 0)),
            pl.BlockSpec((BN_ROWS, DEC), lambda n: (n, 0)),
            pl.BlockSpec((1, DEC), lambda n: (0, 0)),
            pl.BlockSpec((1, DEC), lambda n: (0, 0)),
            pl.BlockSpec((BN_ROWS, 128), lambda n: (n, 0)),
            pl.BlockSpec((BN_ROWS, 128), lambda n: (n, 0)),
        ],
        out_shape=[
            jax.ShapeDtypeStruct((3, N, 128), jnp.float32),
            jax.ShapeDtypeStruct((N, DEC), jnp.float32),
            jax.ShapeDtypeStruct((1, DEC), jnp.float32),
            jax.ShapeDtypeStruct((1, DEC), jnp.float32),
            jax.ShapeDtypeStruct((N, 128), jnp.float32),
            jax.ShapeDtypeStruct((N, 128), jnp.float32),
        ],
    )(s2, fw3, fb, dw1, db1, cw1, cb1, cw2, cb2)


def _var_body(h1_ref, mean_ref, out_ref):
    @pl.when(pl.program_id(0) == 0)
    def _():
        out_ref[...] = jnp.zeros_like(out_ref)
    d = h1_ref[...] - mean_ref[...]
    out_ref[...] += jnp.sum(d * d, axis=0, keepdims=True)


def _var(h1, mean):
    return pl.pallas_call(
        _var_body,
        grid=(N // BN_ROWS,),
        in_specs=[
            pl.BlockSpec((BN_ROWS, DEC), lambda n: (n, 0)),
            pl.BlockSpec((1, DEC), lambda n: (0, 0)),
        ],
        out_specs=pl.BlockSpec((1, DEC), lambda n: (0, 0)),
        out_shape=jax.ShapeDtypeStruct((1, DEC), jnp.float32),
    )(h1, mean)


def _dec_body(h1_ref, sc_ref, sh_ref, w2_ref, b2_ref, wh_ref, bh_ref,
              mu_ref, th_ref, pi_ref):
    hn = h1_ref[...] * sc_ref[...] + sh_ref[...]
    h2 = jax.nn.relu(
        jax.lax.dot_general(hn, w2_ref[...], (((1,), (0,)), ((), ())),
                            preferred_element_type=jnp.float32) + b2_ref[...])
    heads = jax.lax.dot_general(
        h2, wh_ref[...], (((1,), (0,)), ((), ())),
        preferred_element_type=jnp.float32) + bh_ref[...]
    m = heads[:, :G]
    t = heads[:, G:2 * G]
    q = heads[:, 2 * G:]
    mu_ref[...] = jnp.exp(jnp.clip(m, -15.0, 15.0))
    sp = jnp.log1p(jnp.exp(-jnp.abs(t))) + jnp.maximum(t, 0.0)
    th_ref[...] = jnp.clip(sp, 1e-4, 1e4)
    pi_ref[...] = jax.nn.sigmoid(q)


def _dec(h1, scale, shift, w2, b2, wh, bh):
    return pl.pallas_call(
        _dec_body,
        grid=(N // BN_ROWS,),
        in_specs=[
            pl.BlockSpec((BN_ROWS, DEC), lambda n: (n, 0)),
            pl.BlockSpec((1, DEC), lambda n: (0, 0)),
            pl.BlockSpec((1, DEC), lambda n: (0, 0)),
            pl.BlockSpec((DEC, DEC), lambda n: (0, 0)),
            pl.BlockSpec((1, DEC), lambda n: (0, 0)),
            pl.BlockSpec((DEC, 3 * G), lambda n: (0, 0)),
            pl.BlockSpec((1, 3 * G), lambda n: (0, 0)),
        ],
        out_specs=[
            pl.BlockSpec((BN_ROWS, G), lambda n: (n, 0)),
            pl.BlockSpec((BN_ROWS, G), lambda n: (n, 0)),
            pl.BlockSpec((BN_ROWS, G), lambda n: (n, 0)),
        ],
        out_shape=[
            jax.ShapeDtypeStruct((N, G), jnp.float32),
            jax.ShapeDtypeStruct((N, G), jnp.float32),
            jax.ShapeDtypeStruct((N, G), jnp.float32),
        ],
    )(h1, scale, shift, w2, b2, wh, bh)


def _ahat_body(zi_ref, zj_ref, out_ref):
    ip = jax.lax.dot_general(
        zi_ref[...], zj_ref[...], (((1,), (1,)), ((), ())),
        preferred_element_type=jnp.float32)
    ip = jnp.clip(ip, -10.0, 10.0)
    a = jax.nn.sigmoid(ip)
    out_ref[...] = jnp.clip(a, 1e-7, 1.0 - 1e-7)


def _ahat(zn):
    n = zn.shape[0]
    bm = 1024
    bn = 1024
    grid = (pl.cdiv(n, bm), pl.cdiv(n, bn))
    return pl.pallas_call(
        _ahat_body,
        grid=grid,
        in_specs=[
            pl.BlockSpec((bm, L), lambda i, j: (i, 0)),
            pl.BlockSpec((bn, L), lambda i, j: (j, 0)),
        ],
        out_specs=pl.BlockSpec((bm, bn), lambda i, j: (i, j)),
        out_shape=jax.ShapeDtypeStruct((n, n), jnp.float32),
    )(zn, zn)


def kernel(x, ei_knn, ei_mnn, ei_cluster, w_knn, w_mnn, w_cluster, params):
    p = params
    eis = [ei_knn, ei_mnn, ei_cluster]
    ws = [w_knn, w_mnn, w_cluster]
    names = ['knn', 'mnn', 'cluster']
    zeros = jnp.zeros((ROWS_PER_SUB, 128), jnp.float32)

    w1cat = jnp.concatenate([p[n_ + '_W1'] for n_ in names], axis=1)
    b1cat = jnp.concatenate([p[n_ + '_b1'] for n_ in names]).reshape(12, 128)
    tables1 = _enc1(x, w1cat, b1cat).reshape(12 * N, 128)

    pk1, w1 = _pad_edges(eis, ws, NSUB, 90, 79)
    s1 = _make_spmm(12, 4, 1, 90, 79, 16, False)(tables1, pk1, w1, zeros)
    s1 = s1.reshape(12, NP, 128)[:, :N]

    w2s = jnp.stack([p[n_ + '_W2'].reshape(4, 128, 128) for n_ in names])
    b2s = jnp.stack([p[n_ + '_b2'].reshape(1, 128) for n_ in names])
    tables2 = _enc2(s1, w2s, b2s).reshape(3 * N, 128)
    tables2 = jnp.concatenate([tables2, tables2], axis=0)

    pk2, w2 = _pad_edges(eis, ws, 2 * NSUB, 46, 41)
    s2 = _make_spmm(6, 2, 2, 46, 41, 32, True)(tables2, pk2, w2, zeros)
    s2 = s2.reshape(6, NP, 128)[:, :N]

    z3, h1, hsum, hssq, Z_final, Zn = _fuse(
        s2,
        p['fuse_W'].reshape(3, 128, 128), p['fuse_b'].reshape(1, 128),
        p['dec_W1'], p['dec_b1'].reshape(1, DEC),
        p['cv_W1'].reshape(2, 128, 128), p['cv_b1'].reshape(1, 128),
        p['cv_W2'], p['cv_b2'].reshape(1, 128))
    Z_knn, Z_mnn, Z_cluster = z3[0], z3[1], z3[2]

    del hssq
    mean = hsum / N
    var = _var(h1, mean) / N
    scale = p['bn_gamma'].reshape(1, DEC) / jnp.sqrt(var + EPS)
    shift = p['bn_beta'].reshape(1, DEC) - mean * scale

    wh = jnp.concatenate([p['mu_W'], p['th_W'], p['pi_W']], axis=1)
    bh = jnp.concatenate(
        [p['mu_b'], p['th_b'], p['pi_b']]).reshape(1, 3 * G)
    mu, theta, pi = _dec(h1, scale, shift, p['dec_W2'],
                         p['dec_b2'].reshape(1, DEC), wh, bh)

    A_hat = _ahat(Zn)
    return mu, theta, pi, A_hat, Z_final, Z_knn, Z_mnn, Z_cluster
